# Initial kernel scaffold; baseline (speedup 1.0000x reference)
import jax
import jax.numpy as jnp
from jax.experimental import pallas as pl


def kernel(x, edge_index, W, att_src, att_dst, bias, alpha_train, W_out, b_out):
    raise NotImplementedError("write your pallas kernel here")


# speedup vs baseline: 21.9133x; 21.9133x over previous
import functools

import jax
import jax.numpy as jnp
from jax import lax
from jax.experimental import pallas as pl
from jax.experimental.pallas import tpu as pltpu
from jax.experimental.pallas import tpu_sc as plsc

EPS = 0.1
RB = 512
K = 80
NSC = 2
NTILES = 16
NW = NSC * NTILES


def _round_up(v, m):
    return (v + m - 1) // m * m


def _dense_tail(h_blk, att_s, att_d):
    a_s = jnp.sum(h_blk * att_s[None, :], axis=1, keepdims=True)
    a_d = jnp.sum(h_blk * att_d[None, :], axis=1, keepdims=True)
    return a_s, a_d


def _prep1_body(x_ref, w_ref, as_ref, ad_ref, h_ref, asrc_ref, adst_ref):
    h = jnp.dot(x_ref[...], w_ref[...], preferred_element_type=jnp.float32)
    h_ref[...] = h
    a_s, a_d = _dense_tail(h, as_ref[...], ad_ref[...])
    asrc_ref[...] = a_s
    adst_ref[...] = a_d


def _gat_from_parts(scat, den, bias):
    num = scat[0] + scat[1]
    dn = den[0] + den[1] + 1e-16
    return num / dn[:, None] + bias[None, :]


def _prep_body(coef, x_ref, zp_ref, scat_ref, den_ref, alpha_ref, w_ref,
               as_ref, ad_ref, bias_ref,
               k_ref, z_ref, h_ref, asrc_ref, adst_ref):
    a = jax.nn.sigmoid(alpha_ref[0, 0])
    gat = _gat_from_parts(scat_ref[...], den_ref[...], bias_ref[...])
    k = a * (gat - zp_ref[...])
    k_ref[...] = k
    z = x_ref[...] + coef * k
    z_ref[...] = z
    h = jnp.dot(z, w_ref[...], preferred_element_type=jnp.float32)
    h_ref[...] = h
    a_s, a_d = _dense_tail(h, as_ref[...], ad_ref[...])
    asrc_ref[...] = a_s
    adst_ref[...] = a_d


def _final_body(x_ref, zp_ref, scat_ref, den_ref, alpha_ref, bias_ref,
                k1_ref, k2_ref, k3_ref, wout_ref, bout_ref, out_ref):
    a = jax.nn.sigmoid(alpha_ref[0, 0])
    gat = _gat_from_parts(scat_ref[...], den_ref[...], bias_ref[...])
    k4 = a * (gat - zp_ref[...])
    z = x_ref[...] + (EPS / 6.0) * (k1_ref[...] + 2.0 * k2_ref[...]
                                    + 2.0 * k3_ref[...] + k4)
    y = jnp.tanh(z)
    out_ref[...] = jnp.dot(y, wout_ref[...], preferred_element_type=jnp.float32) \
        + bout_ref[...][None, :]


def _mat_spec(i_map=lambda i: (i, 0), blk=None):
    return pl.BlockSpec(blk, i_map)


def _make_tc_calls(P, D):
    grid = (P // RB,)
    row = pl.BlockSpec((RB, D), lambda i: (i, 0))
    full_w = pl.BlockSpec((D, D), lambda i: (0, 0))
    vec = pl.BlockSpec((D,), lambda i: (0,))
    col = pl.BlockSpec((RB, 1), lambda i: (i, 0))
    scat_s = pl.BlockSpec((NSC, RB, D), lambda i: (0, i, 0))
    den_s = pl.BlockSpec((NSC, RB), lambda i: (0, i))
    scal = pl.BlockSpec((1, 1), lambda i: (0, 0))

    f32 = jnp.float32
    prep1 = pl.pallas_call(
        _prep1_body, grid=grid,
        in_specs=[row, full_w, vec, vec],
        out_specs=[row, col, col],
        out_shape=[jax.ShapeDtypeStruct((P, D), f32),
                   jax.ShapeDtypeStruct((P, 1), f32),
                   jax.ShapeDtypeStruct((P, 1), f32)],
    )

    def prep(coef):
        return pl.pallas_call(
            functools.partial(_prep_body, coef), grid=grid,
            in_specs=[row, row, scat_s, den_s, scal, full_w, vec, vec, vec],
            out_specs=[row, row, row, col, col],
            out_shape=[jax.ShapeDtypeStruct((P, D), f32)] * 3
            + [jax.ShapeDtypeStruct((P, 1), f32)] * 2,
        )

    final = pl.pallas_call(
        _final_body, grid=grid,
        in_specs=[row, row, scat_s, den_s, scal, vec, row, row, row, full_w, vec],
        out_specs=row,
        out_shape=jax.ShapeDtypeStruct((P, D), f32),
    )
    return prep1, prep, final


def _make_sc_stage(P, D, Epad):
    ET = Epad // NW
    NB = ET // K
    RPT = P // NTILES
    assert RPT % K == 0
    NZ = RPT // K
    mesh = plsc.VectorSubcoreMesh(core_axis_name="c", subcore_axis_name="s")
    f32 = jnp.float32

    @functools.partial(
        pl.kernel, mesh=mesh,
        compiler_params=pltpu.CompilerParams(needs_layout_passes=False),
        out_type=[jax.ShapeDtypeStruct((NSC, P, D), f32),
                  jax.ShapeDtypeStruct((NSC, P), f32)],
        scratch_types=[
            pltpu.VMEM((P,), f32),
            pltpu.VMEM((P,), f32),
            pltpu.VMEM((K,), jnp.int32),
            pltpu.VMEM((K,), jnp.int32),
            pltpu.VMEM((K, D), f32),
            pltpu.VMEM((K,), f32),
            pltpu.VMEM_SHARED((P, D), f32),
            pltpu.VMEM_SHARED((P,), f32),
            pltpu.SemaphoreType.DMA,
            pltpu.SemaphoreType.DMA,
            pltpu.SemaphoreType.DMA,
        ],
    )
    def sc_stage(src_h, dst_h, h_h, asv_h, adv_h, scat_o, den_o,
                 as_v, ad_v, srcb, dstb, rows, exb, scat_s, den_s,
                 sem_g, sem_s, sem_d):
        core = lax.axis_index("c")
        sub = lax.axis_index("s")
        wid = core * NTILES + sub

        pltpu.sync_copy(asv_h, as_v)
        pltpu.sync_copy(adv_h, ad_v)

        def zrow(i, _):
            for c in range(D // 16):
                rows[i, pl.ds(c * 16, 16)] = jnp.zeros((16,), f32)
            return _
        lax.fori_loop(0, K, zrow, None)
        for c in range(K // 16):
            exb[pl.ds(c * 16, 16)] = jnp.zeros((16,), f32)

        zbase = sub * RPT

        def zcopy(j, _):
            pltpu.sync_copy(rows, scat_s.at[pl.ds(zbase + j * K, K)])
            pltpu.sync_copy(exb, den_s.at[pl.ds(zbase + j * K, K)])
            return _
        lax.fori_loop(0, NZ, zcopy, None)

        def amax_body(i, acc):
            return jnp.maximum(acc, as_v[pl.ds(i * 16, 16)])
        acc = lax.fori_loop(0, P // 16, amax_body, jnp.zeros((16,), f32))
        big_a = plsc.cummax(acc)[15]

        plsc.subcore_barrier()

        ebase = wid * ET

        def block(b, _):
            off = ebase + b * K
            pltpu.sync_copy(src_h.at[pl.ds(off, K)], srcb)
            pltpu.sync_copy(dst_h.at[pl.ds(off, K)], dstb)
            pltpu.async_copy(h_h.at[srcb], rows, sem_g).wait()
            for g in range(K // 16):
                sl = pl.ds(g * 16, 16)
                s16 = plsc.load_gather(as_v, [srcb[sl]])
                d16 = plsc.load_gather(ad_v, [dstb[sl]])
                e = s16 + d16
                e = jnp.where(e > 0.0, e, 0.2 * e)
                mt = jnp.maximum(big_a + d16, 0.0)
                ex = jnp.exp(e - mt)
                exb[sl] = ex
                for i in range(16):
                    w = ex[i]
                    for c in range(D // 16):
                        cs = pl.ds(c * 16, 16)
                        j = g * 16 + i
                        rows[j, cs] = rows[j, cs] * w
            pltpu.async_copy(rows, scat_s.at[dstb], sem_s, add=True).wait()
            pltpu.async_copy(exb, den_s.at[dstb], sem_d, add=True).wait()
            return _
        lax.fori_loop(0, NB, block, None)

        plsc.subcore_barrier()

        pltpu.sync_copy(scat_s.at[pl.ds(zbase, RPT)],
                        scat_o.at[core, pl.ds(zbase, RPT)])
        pltpu.sync_copy(den_s.at[pl.ds(zbase, RPT)],
                        den_o.at[core, pl.ds(zbase, RPT)])

    return sc_stage


def kernel(x, edge_index, W, att_src, att_dst, bias, alpha_train, W_out, b_out):
    N, D = x.shape
    E = edge_index.shape[1]
    P = _round_up(N, 5120)

    ei = edge_index.astype(jnp.int32)
    loops = jnp.arange(N, dtype=jnp.int32)
    ET = E + N
    Epad = _round_up(ET, NW * K)
    fill = jnp.full((Epad - ET,), N, dtype=jnp.int32)
    src = jnp.concatenate([ei[0], loops, fill])
    dst = jnp.concatenate([ei[1], loops, fill])

    xp = jnp.pad(x, ((0, P - N), (0, 0)))
    alpha2d = alpha_train.reshape(1, 1)

    prep1, prep, final = _make_tc_calls(P, D)
    sc_stage = _make_sc_stage(P, D, Epad)

    h1, as1, ad1 = prep1(xp, W, att_src, att_dst)
    sc1, dn1 = sc_stage(src, dst, h1, as1.reshape(P), ad1.reshape(P))

    k1, z2, h2, as2, ad2 = prep(0.5 * EPS)(xp, xp, sc1, dn1, alpha2d, W,
                                           att_src, att_dst, bias)
    sc2, dn2 = sc_stage(src, dst, h2, as2.reshape(P), ad2.reshape(P))

    k2, z3, h3, as3, ad3 = prep(0.5 * EPS)(xp, z2, sc2, dn2, alpha2d, W,
                                           att_src, att_dst, bias)
    sc3, dn3 = sc_stage(src, dst, h3, as3.reshape(P), ad3.reshape(P))

    k3, z4, h4, as4, ad4 = prep(EPS)(xp, z3, sc3, dn3, alpha2d, W,
                                     att_src, att_dst, bias)
    sc4, dn4 = sc_stage(src, dst, h4, as4.reshape(P), ad4.reshape(P))

    out = final(xp, z4, sc4, dn4, alpha2d, bias, k1, k2, k3, W_out, b_out)
    return out[:N]


# speedup vs baseline: 24.6554x; 1.1251x over previous
import functools

import jax
import jax.numpy as jnp
from jax import lax
from jax.experimental import pallas as pl
from jax.experimental.pallas import tpu as pltpu
from jax.experimental.pallas import tpu_sc as plsc

EPS = 0.1
RB = 512
K = 80
NSC = 2
NTILES = 16
NW = NSC * NTILES


def _round_up(v, m):
    return (v + m - 1) // m * m


def _dense_tail(h_blk, att_s, att_d):
    a_s = jnp.sum(h_blk * att_s[None, :], axis=1, keepdims=True)
    a_d = jnp.sum(h_blk * att_d[None, :], axis=1, keepdims=True)
    return a_s, a_d


def _prep1_body(x_ref, w_ref, as_ref, ad_ref, h_ref, asrc_ref, adst_ref):
    h = jnp.dot(x_ref[...], w_ref[...], preferred_element_type=jnp.float32)
    h_ref[...] = h
    a_s, a_d = _dense_tail(h, as_ref[...], ad_ref[...])
    asrc_ref[...] = a_s
    adst_ref[...] = a_d


def _gat_from_parts(scat, den, bias):
    num = scat[0] + scat[1]
    dn = den[0] + den[1] + 1e-16
    return num / dn[:, None] + bias[None, :]


def _prep_body(coef, x_ref, zp_ref, scat_ref, den_ref, alpha_ref, w_ref,
               as_ref, ad_ref, bias_ref,
               k_ref, z_ref, h_ref, asrc_ref, adst_ref):
    a = jax.nn.sigmoid(alpha_ref[0, 0])
    gat = _gat_from_parts(scat_ref[...], den_ref[...], bias_ref[...])
    k = a * (gat - zp_ref[...])
    k_ref[...] = k
    z = x_ref[...] + coef * k
    z_ref[...] = z
    h = jnp.dot(z, w_ref[...], preferred_element_type=jnp.float32)
    h_ref[...] = h
    a_s, a_d = _dense_tail(h, as_ref[...], ad_ref[...])
    asrc_ref[...] = a_s
    adst_ref[...] = a_d


def _final_body(x_ref, zp_ref, scat_ref, den_ref, alpha_ref, bias_ref,
                k1_ref, k2_ref, k3_ref, wout_ref, bout_ref, out_ref):
    a = jax.nn.sigmoid(alpha_ref[0, 0])
    gat = _gat_from_parts(scat_ref[...], den_ref[...], bias_ref[...])
    k4 = a * (gat - zp_ref[...])
    z = x_ref[...] + (EPS / 6.0) * (k1_ref[...] + 2.0 * k2_ref[...]
                                    + 2.0 * k3_ref[...] + k4)
    y = jnp.tanh(z)
    out_ref[...] = jnp.dot(y, wout_ref[...], preferred_element_type=jnp.float32) \
        + bout_ref[...][None, :]


def _mat_spec(i_map=lambda i: (i, 0), blk=None):
    return pl.BlockSpec(blk, i_map)


def _make_tc_calls(P, D):
    grid = (P // RB,)
    row = pl.BlockSpec((RB, D), lambda i: (i, 0))
    full_w = pl.BlockSpec((D, D), lambda i: (0, 0))
    vec = pl.BlockSpec((D,), lambda i: (0,))
    col = pl.BlockSpec((RB, 1), lambda i: (i, 0))
    scat_s = pl.BlockSpec((NSC, RB, D), lambda i: (0, i, 0))
    den_s = pl.BlockSpec((NSC, RB), lambda i: (0, i))
    scal = pl.BlockSpec((1, 1), lambda i: (0, 0))

    f32 = jnp.float32
    prep1 = pl.pallas_call(
        _prep1_body, grid=grid,
        in_specs=[row, full_w, vec, vec],
        out_specs=[row, col, col],
        out_shape=[jax.ShapeDtypeStruct((P, D), f32),
                   jax.ShapeDtypeStruct((P, 1), f32),
                   jax.ShapeDtypeStruct((P, 1), f32)],
    )

    def prep(coef):
        return pl.pallas_call(
            functools.partial(_prep_body, coef), grid=grid,
            in_specs=[row, row, scat_s, den_s, scal, full_w, vec, vec, vec],
            out_specs=[row, row, row, col, col],
            out_shape=[jax.ShapeDtypeStruct((P, D), f32)] * 3
            + [jax.ShapeDtypeStruct((P, 1), f32)] * 2,
        )

    final = pl.pallas_call(
        _final_body, grid=grid,
        in_specs=[row, row, scat_s, den_s, scal, vec, row, row, row, full_w, vec],
        out_specs=row,
        out_shape=jax.ShapeDtypeStruct((P, D), f32),
    )
    return prep1, prep, final


def _make_sc_stage(P, D, Epad):
    ET = Epad // NW
    NB = ET // K
    assert NB % 2 == 0
    RPT = P // NTILES
    assert RPT % K == 0
    NZ = RPT // K
    mesh = plsc.VectorSubcoreMesh(core_axis_name="c", subcore_axis_name="s")
    f32 = jnp.float32

    @functools.partial(
        pl.kernel, mesh=mesh,
        compiler_params=pltpu.CompilerParams(needs_layout_passes=False),
        out_type=[jax.ShapeDtypeStruct((NSC, P, D), f32),
                  jax.ShapeDtypeStruct((NSC, P), f32)],
        scratch_types=[
            pltpu.VMEM((P,), f32),
            pltpu.VMEM((P,), f32),
            pltpu.VMEM((K,), jnp.int32),
            pltpu.VMEM((K,), jnp.int32),
            pltpu.VMEM((K,), jnp.int32),
            pltpu.VMEM((K,), jnp.int32),
            pltpu.VMEM((K, D), f32),
            pltpu.VMEM((K, D), f32),
            pltpu.VMEM((K,), f32),
            pltpu.VMEM((K,), f32),
            pltpu.VMEM_SHARED((P, D), f32),
            pltpu.VMEM_SHARED((P,), f32),
            pltpu.SemaphoreType.DMA,
            pltpu.SemaphoreType.DMA,
            pltpu.SemaphoreType.DMA,
            pltpu.SemaphoreType.DMA,
            pltpu.SemaphoreType.DMA,
            pltpu.SemaphoreType.DMA,
        ],
    )
    def sc_stage(src_h, dst_h, h_h, asv_h, adv_h, scat_o, den_o,
                 as_v, ad_v, srcb0, srcb1, dstb0, dstb1, rows0, rows1,
                 exb0, exb1, scat_s, den_s,
                 sem_g0, sem_g1, sem_s0, sem_s1, sem_d0, sem_d1):
        core = lax.axis_index("c")
        sub = lax.axis_index("s")
        wid = core * NTILES + sub
        sem_g = (sem_g0, sem_g1)
        sem_s = (sem_s0, sem_s1)
        sem_d = (sem_d0, sem_d1)
        srcb = (srcb0, srcb1)
        dstb = (dstb0, dstb1)
        rows = (rows0, rows1)
        exb = (exb0, exb1)

        pltpu.sync_copy(asv_h, as_v)
        pltpu.sync_copy(adv_h, ad_v)

        def zrow(i, _):
            for c in range(D // 16):
                rows0[i, pl.ds(c * 16, 16)] = jnp.zeros((16,), f32)
            return _
        lax.fori_loop(0, K, zrow, None)
        for c in range(K // 16):
            exb0[pl.ds(c * 16, 16)] = jnp.zeros((16,), f32)

        zbase = sub * RPT

        def zcopy(j, _):
            pltpu.sync_copy(rows0, scat_s.at[pl.ds(zbase + j * K, K)])
            pltpu.sync_copy(exb0, den_s.at[pl.ds(zbase + j * K, K)])
            return _
        lax.fori_loop(0, NZ, zcopy, None)

        def amax_body(i, acc):
            return jnp.maximum(acc, as_v[pl.ds(i * 16, 16)])
        acc = lax.fori_loop(0, P // 16, amax_body, jnp.zeros((16,), f32))
        big_a = plsc.cummax(acc)[15]

        plsc.subcore_barrier()

        ebase = wid * ET

        def stage_idx(par, b):
            off = ebase + b * K
            pltpu.sync_copy(src_h.at[pl.ds(off, K)], srcb[par])
            pltpu.sync_copy(dst_h.at[pl.ds(off, K)], dstb[par])

        for par in (0, 1):
            stage_idx(par, par)
            pltpu.async_copy(h_h.at[srcb[par]], rows[par], sem_g[par])

        def pair(i2, _):
            scats = []
            for par in (0, 1):
                pltpu.make_async_copy(h_h.at[srcb[par]], rows[par],
                                      sem_g[par]).wait()

                def grp(g, _, par=par):
                    sl = pl.ds(g * 16, 16)
                    srcg = srcb[par][sl]
                    dstg = dstb[par][sl]
                    s16 = plsc.load_gather(as_v, [srcg])
                    d16 = plsc.load_gather(ad_v, [dstg])
                    e = s16 + d16
                    e = jnp.where(e > 0.0, e, 0.2 * e)
                    mt = jnp.maximum(big_a + d16, 0.0)
                    ex = jnp.exp(e - mt)
                    exb[par][sl] = ex
                    for i in range(16):
                        w = ex[i]
                        j = g * 16 + i
                        for c in range(D // 16):
                            cs = pl.ds(c * 16, 16)
                            rows[par][j, cs] = rows[par][j, cs] * w
                    return _
                lax.fori_loop(0, K // 16, grp, None)
                s = pltpu.async_copy(rows[par], scat_s.at[dstb[par]],
                                     sem_s[par], add=True)
                d = pltpu.async_copy(exb[par], den_s.at[dstb[par]],
                                     sem_d[par], add=True)
                scats.append((s, d))
            for par in (0, 1):
                b = 2 * i2 + par
                s, d = scats[par]
                s.wait()
                d.wait()
                stage_idx(par, b + 2)
                pltpu.async_copy(h_h.at[srcb[par]], rows[par], sem_g[par])
            return _
        lax.fori_loop(0, NB // 2, pair, None)

        for par in (0, 1):
            pltpu.make_async_copy(h_h.at[srcb[par]], rows[par],
                                  sem_g[par]).wait()

        plsc.subcore_barrier()

        pltpu.sync_copy(scat_s.at[pl.ds(zbase, RPT)],
                        scat_o.at[core, pl.ds(zbase, RPT)])
        pltpu.sync_copy(den_s.at[pl.ds(zbase, RPT)],
                        den_o.at[core, pl.ds(zbase, RPT)])

    return sc_stage


def kernel(x, edge_index, W, att_src, att_dst, bias, alpha_train, W_out, b_out):
    N, D = x.shape
    E = edge_index.shape[1]
    P = _round_up(N, 5120)

    ei = edge_index.astype(jnp.int32)
    loops = jnp.arange(N, dtype=jnp.int32)
    ETOT = E + N
    Epad = _round_up(ETOT, NW * 2 * K)
    fill = jnp.full((Epad + 2 * K - ETOT,), N, dtype=jnp.int32)
    src = jnp.concatenate([ei[0], loops, fill])
    dst = jnp.concatenate([ei[1], loops, fill])

    xp = jnp.pad(x, ((0, P - N), (0, 0)))
    alpha2d = alpha_train.reshape(1, 1)

    prep1, prep, final = _make_tc_calls(P, D)
    sc_stage = _make_sc_stage(P, D, Epad)

    h1, as1, ad1 = prep1(xp, W, att_src, att_dst)
    sc1, dn1 = sc_stage(src, dst, h1, as1.reshape(P), ad1.reshape(P))

    k1, z2, h2, as2, ad2 = prep(0.5 * EPS)(xp, xp, sc1, dn1, alpha2d, W,
                                           att_src, att_dst, bias)
    sc2, dn2 = sc_stage(src, dst, h2, as2.reshape(P), ad2.reshape(P))

    k2, z3, h3, as3, ad3 = prep(0.5 * EPS)(xp, z2, sc2, dn2, alpha2d, W,
                                           att_src, att_dst, bias)
    sc3, dn3 = sc_stage(src, dst, h3, as3.reshape(P), ad3.reshape(P))

    k3, z4, h4, as4, ad4 = prep(EPS)(xp, z3, sc3, dn3, alpha2d, W,
                                     att_src, att_dst, bias)
    sc4, dn4 = sc_stage(src, dst, h4, as4.reshape(P), ad4.reshape(P))

    out = final(xp, z4, sc4, dn4, alpha2d, bias, k1, k2, k3, W_out, b_out)
    return out[:N]


# speedup vs baseline: 29.7474x; 1.2065x over previous
import functools

import jax
import jax.numpy as jnp
from jax import lax
from jax.experimental import pallas as pl
from jax.experimental.pallas import tpu as pltpu
from jax.experimental.pallas import tpu_sc as plsc

EPS = 0.1
RB = 512
K = 64
NSC = 2
NTILES = 16
NW = NSC * NTILES


def _round_up(v, m):
    return (v + m - 1) // m * m


def _dense_tail(h_blk, att_s, att_d):
    a_s = jnp.sum(h_blk * att_s[None, :], axis=1, keepdims=True)
    a_d = jnp.sum(h_blk * att_d[None, :], axis=1, keepdims=True)
    return a_s, a_d


def _prep1_body(x_ref, w_ref, as_ref, ad_ref, h_ref, asrc_ref, adst_ref):
    h = jnp.dot(x_ref[...], w_ref[...], preferred_element_type=jnp.float32)
    h_ref[...] = h
    a_s, a_d = _dense_tail(h, as_ref[...], ad_ref[...])
    asrc_ref[...] = a_s
    adst_ref[...] = a_d


def _gat_from_parts(scat, den, bias):
    num = scat[0] + scat[1]
    dn = den[0] + den[1] + 1e-16
    return num / dn[:, None] + bias[None, :]


def _prep_body(coef, x_ref, zp_ref, scat_ref, den_ref, alpha_ref, w_ref,
               as_ref, ad_ref, bias_ref,
               k_ref, z_ref, h_ref, asrc_ref, adst_ref):
    a = jax.nn.sigmoid(alpha_ref[0, 0])
    gat = _gat_from_parts(scat_ref[...], den_ref[...], bias_ref[...])
    k = a * (gat - zp_ref[...])
    k_ref[...] = k
    z = x_ref[...] + coef * k
    z_ref[...] = z
    h = jnp.dot(z, w_ref[...], preferred_element_type=jnp.float32)
    h_ref[...] = h
    a_s, a_d = _dense_tail(h, as_ref[...], ad_ref[...])
    asrc_ref[...] = a_s
    adst_ref[...] = a_d


def _final_body(x_ref, zp_ref, scat_ref, den_ref, alpha_ref, bias_ref,
                k1_ref, k2_ref, k3_ref, wout_ref, bout_ref, out_ref):
    a = jax.nn.sigmoid(alpha_ref[0, 0])
    gat = _gat_from_parts(scat_ref[...], den_ref[...], bias_ref[...])
    k4 = a * (gat - zp_ref[...])
    z = x_ref[...] + (EPS / 6.0) * (k1_ref[...] + 2.0 * k2_ref[...]
                                    + 2.0 * k3_ref[...] + k4)
    y = jnp.tanh(z)
    out_ref[...] = jnp.dot(y, wout_ref[...], preferred_element_type=jnp.float32) \
        + bout_ref[...][None, :]


def _mat_spec(i_map=lambda i: (i, 0), blk=None):
    return pl.BlockSpec(blk, i_map)


def _make_tc_calls(P, D):
    grid = (P // RB,)
    row = pl.BlockSpec((RB, D), lambda i: (i, 0))
    full_w = pl.BlockSpec((D, D), lambda i: (0, 0))
    vec = pl.BlockSpec((D,), lambda i: (0,))
    col = pl.BlockSpec((RB, 1), lambda i: (i, 0))
    scat_s = pl.BlockSpec((NSC, RB, D), lambda i: (0, i, 0))
    den_s = pl.BlockSpec((NSC, RB), lambda i: (0, i))
    scal = pl.BlockSpec((1, 1), lambda i: (0, 0))

    f32 = jnp.float32
    prep1 = pl.pallas_call(
        _prep1_body, grid=grid,
        in_specs=[row, full_w, vec, vec],
        out_specs=[row, col, col],
        out_shape=[jax.ShapeDtypeStruct((P, D), f32),
                   jax.ShapeDtypeStruct((P, 1), f32),
                   jax.ShapeDtypeStruct((P, 1), f32)],
    )

    def prep(coef):
        return pl.pallas_call(
            functools.partial(_prep_body, coef), grid=grid,
            in_specs=[row, row, scat_s, den_s, scal, full_w, vec, vec, vec],
            out_specs=[row, row, row, col, col],
            out_shape=[jax.ShapeDtypeStruct((P, D), f32)] * 3
            + [jax.ShapeDtypeStruct((P, 1), f32)] * 2,
        )

    final = pl.pallas_call(
        _final_body, grid=grid,
        in_specs=[row, row, scat_s, den_s, scal, vec, row, row, row, full_w, vec],
        out_specs=row,
        out_shape=jax.ShapeDtypeStruct((P, D), f32),
    )
    return prep1, prep, final


def _make_sc_stage(P, D, Epad):
    ET = Epad // NW
    NB = ET // K
    assert NB % 3 == 0
    RPT = P // NTILES
    assert RPT % K == 0
    NZ = RPT // K
    mesh = plsc.VectorSubcoreMesh(core_axis_name="c", subcore_axis_name="s")
    f32 = jnp.float32

    @functools.partial(
        pl.kernel, mesh=mesh,
        compiler_params=pltpu.CompilerParams(needs_layout_passes=False),
        out_type=[jax.ShapeDtypeStruct((NSC, P, D), f32),
                  jax.ShapeDtypeStruct((NSC, P), f32)],
        scratch_types=[
            pltpu.VMEM((P,), f32),
            pltpu.VMEM((P,), f32),
            tuple(pltpu.VMEM((K,), jnp.int32) for _ in range(3)),
            tuple(pltpu.VMEM((K,), jnp.int32) for _ in range(3)),
            tuple(pltpu.VMEM((K, D), f32) for _ in range(3)),
            tuple(pltpu.VMEM((K,), f32) for _ in range(3)),
            pltpu.VMEM_SHARED((P, D), f32),
            pltpu.VMEM_SHARED((P,), f32),
            tuple(pltpu.SemaphoreType.DMA for _ in range(3)),
            tuple(pltpu.SemaphoreType.DMA for _ in range(3)),
            tuple(pltpu.SemaphoreType.DMA for _ in range(3)),
        ],
    )
    def sc_stage(src_h, dst_h, h_h, asv_h, adv_h, scat_o, den_o,
                 as_v, ad_v, srcb, dstb, rows, exb, scat_s, den_s,
                 sem_g, sem_s, sem_d):
        core = lax.axis_index("c")
        sub = lax.axis_index("s")
        wid = core * NTILES + sub

        pltpu.sync_copy(asv_h, as_v)
        pltpu.sync_copy(adv_h, ad_v)

        def zrow(i, _):
            for c in range(D // 16):
                rows[0][i, pl.ds(c * 16, 16)] = jnp.zeros((16,), f32)
            return _
        lax.fori_loop(0, K, zrow, None)
        for c in range(K // 16):
            exb[0][pl.ds(c * 16, 16)] = jnp.zeros((16,), f32)

        zbase = sub * RPT

        def zcopy(j, _):
            pltpu.sync_copy(rows[0], scat_s.at[pl.ds(zbase + j * K, K)])
            pltpu.sync_copy(exb[0], den_s.at[pl.ds(zbase + j * K, K)])
            return _
        lax.fori_loop(0, NZ, zcopy, None)

        def amax_body(i, acc):
            return jnp.maximum(acc, as_v[pl.ds(i * 16, 16)])
        acc = lax.fori_loop(0, P // 16, amax_body, jnp.zeros((16,), f32))
        big_a = plsc.cummax(acc)[15]

        plsc.subcore_barrier()

        ebase = wid * ET

        def stage_idx(islot, b):
            off = ebase + b * K
            pltpu.sync_copy(src_h.at[pl.ds(off, K)], srcb[islot])
            pltpu.sync_copy(dst_h.at[pl.ds(off, K)], dstb[islot])

        def issue_gather(slot):
            pltpu.async_copy(h_h.at[srcb[slot]], rows[slot], sem_g[slot])

        def wait_gather(slot):
            pltpu.make_async_copy(h_h.at[srcb[slot]], rows[slot],
                                  sem_g[slot]).wait()

        for b in range(2):
            stage_idx(b, b)
            issue_gather(b)

        def trip(i3, _):
            sdescs = []
            for u in range(3):
                b = 3 * i3 + u
                slot = u
                wait_gather(slot)

                def grp(g, _, slot=slot):
                    sl = pl.ds(g * 16, 16)
                    srcg = srcb[slot][sl]
                    dstg = dstb[slot][sl]
                    s16 = plsc.load_gather(as_v, [srcg])
                    d16 = plsc.load_gather(ad_v, [dstg])
                    e = s16 + d16
                    e = jnp.where(e > 0.0, e, 0.2 * e)
                    mt = jnp.maximum(big_a + d16, 0.0)
                    ex = jnp.exp(e - mt)
                    exb[slot][sl] = ex
                    for i in range(16):
                        w = ex[i]
                        j = g * 16 + i
                        for c in range(D // 16):
                            cs = pl.ds(c * 16, 16)
                            rows[slot][j, cs] = rows[slot][j, cs] * w
                    return _
                lax.fori_loop(0, K // 16, grp, None)
                s = pltpu.async_copy(rows[slot], scat_s.at[dstb[slot]],
                                     sem_s[slot], add=True)
                d = pltpu.async_copy(exb[slot], den_s.at[dstb[slot]],
                                     sem_d[slot], add=True)
                sdescs.append((s, d))

                if u >= 1:
                    ps, pd = sdescs[u - 1]
                    ps.wait()
                    pd.wait()
                nslot = (u + 2) % 3
                stage_idx(nslot, b + 2)
                issue_gather(nslot)
            ls, ld = sdescs[2]
            ls.wait()
            ld.wait()
            return _
        lax.fori_loop(0, NB // 3, trip, None)

        for b in (NB, NB + 1):
            wait_gather(b % 3)

        plsc.subcore_barrier()

        pltpu.sync_copy(scat_s.at[pl.ds(zbase, RPT)],
                        scat_o.at[core, pl.ds(zbase, RPT)])
        pltpu.sync_copy(den_s.at[pl.ds(zbase, RPT)],
                        den_o.at[core, pl.ds(zbase, RPT)])

    return sc_stage


def kernel(x, edge_index, W, att_src, att_dst, bias, alpha_train, W_out, b_out):
    N, D = x.shape
    E = edge_index.shape[1]
    P = _round_up(N, 5120)

    ei = edge_index.astype(jnp.int32)
    loops = jnp.arange(N, dtype=jnp.int32)
    ETOT = E + N
    Epad = _round_up(ETOT, NW * 3 * K)
    fill = jnp.full((Epad + 2 * K - ETOT,), N, dtype=jnp.int32)
    src = jnp.concatenate([ei[0], loops, fill])
    dst = jnp.concatenate([ei[1], loops, fill])

    xp = jnp.pad(x, ((0, P - N), (0, 0)))
    alpha2d = alpha_train.reshape(1, 1)

    prep1, prep, final = _make_tc_calls(P, D)
    sc_stage = _make_sc_stage(P, D, Epad)

    h1, as1, ad1 = prep1(xp, W, att_src, att_dst)
    sc1, dn1 = sc_stage(src, dst, h1, as1.reshape(P), ad1.reshape(P))

    k1, z2, h2, as2, ad2 = prep(0.5 * EPS)(xp, xp, sc1, dn1, alpha2d, W,
                                           att_src, att_dst, bias)
    sc2, dn2 = sc_stage(src, dst, h2, as2.reshape(P), ad2.reshape(P))

    k2, z3, h3, as3, ad3 = prep(0.5 * EPS)(xp, z2, sc2, dn2, alpha2d, W,
                                           att_src, att_dst, bias)
    sc3, dn3 = sc_stage(src, dst, h3, as3.reshape(P), ad3.reshape(P))

    k3, z4, h4, as4, ad4 = prep(EPS)(xp, z3, sc3, dn3, alpha2d, W,
                                     att_src, att_dst, bias)
    sc4, dn4 = sc_stage(src, dst, h4, as4.reshape(P), ad4.reshape(P))

    out = final(xp, z4, sc4, dn4, alpha2d, bias, k1, k2, k3, W_out, b_out)
    return out[:N]


# speedup vs baseline: 36.6683x; 1.2327x over previous
import functools

import jax
import jax.numpy as jnp
from jax import lax
from jax.experimental import pallas as pl
from jax.experimental.pallas import tpu as pltpu
from jax.experimental.pallas import tpu_sc as plsc

EPS = 0.1
RB = 512
K = 64
NSC = 2
NTILES = 16
NW = NSC * NTILES


def _round_up(v, m):
    return (v + m - 1) // m * m


def _dense_tail(h_blk, att_s, att_d):
    a_s = jnp.sum(h_blk * att_s[None, :], axis=1, keepdims=True)
    a_d = jnp.sum(h_blk * att_d[None, :], axis=1, keepdims=True)
    return a_s, a_d


def _prep1_body(x_ref, w_ref, as_ref, ad_ref, h_ref, asrc_ref, adst_ref):
    h = jnp.dot(x_ref[...], w_ref[...], preferred_element_type=jnp.float32)
    h_ref[...] = h
    a_s, a_d = _dense_tail(h, as_ref[...], ad_ref[...])
    asrc_ref[...] = a_s
    adst_ref[...] = a_d


def _gat_from_parts(scat, den, bias):
    num = scat[0] + scat[1]
    dn = den[0] + den[1] + 1e-16
    return num / dn[:, None] + bias[None, :]


def _prep_body(coef, x_ref, zp_ref, scat_ref, den_ref, alpha_ref, w_ref,
               as_ref, ad_ref, bias_ref,
               k_ref, z_ref, h_ref, asrc_ref, adst_ref):
    a = jax.nn.sigmoid(alpha_ref[0, 0])
    gat = _gat_from_parts(scat_ref[...], den_ref[...], bias_ref[...])
    k = a * (gat - zp_ref[...])
    k_ref[...] = k
    z = x_ref[...] + coef * k
    z_ref[...] = z
    h = jnp.dot(z, w_ref[...], preferred_element_type=jnp.float32)
    h_ref[...] = h
    a_s, a_d = _dense_tail(h, as_ref[...], ad_ref[...])
    asrc_ref[...] = a_s
    adst_ref[...] = a_d


def _final_body(x_ref, zp_ref, scat_ref, den_ref, alpha_ref, bias_ref,
                k1_ref, k2_ref, k3_ref, wout_ref, bout_ref, out_ref):
    a = jax.nn.sigmoid(alpha_ref[0, 0])
    gat = _gat_from_parts(scat_ref[...], den_ref[...], bias_ref[...])
    k4 = a * (gat - zp_ref[...])
    z = x_ref[...] + (EPS / 6.0) * (k1_ref[...] + 2.0 * k2_ref[...]
                                    + 2.0 * k3_ref[...] + k4)
    y = jnp.tanh(z)
    out_ref[...] = jnp.dot(y, wout_ref[...], preferred_element_type=jnp.float32) \
        + bout_ref[...][None, :]


def _mat_spec(i_map=lambda i: (i, 0), blk=None):
    return pl.BlockSpec(blk, i_map)


def _make_tc_calls(P, D):
    grid = (P // RB,)
    row = pl.BlockSpec((RB, D), lambda i: (i, 0))
    full_w = pl.BlockSpec((D, D), lambda i: (0, 0))
    vec = pl.BlockSpec((D,), lambda i: (0,))
    col = pl.BlockSpec((RB, 1), lambda i: (i, 0))
    scat_s = pl.BlockSpec((NSC, RB, D), lambda i: (0, i, 0))
    den_s = pl.BlockSpec((NSC, RB), lambda i: (0, i))
    scal = pl.BlockSpec((1, 1), lambda i: (0, 0))

    f32 = jnp.float32
    prep1 = pl.pallas_call(
        _prep1_body, grid=grid,
        in_specs=[row, full_w, vec, vec],
        out_specs=[row, col, col],
        out_shape=[jax.ShapeDtypeStruct((P, D), f32),
                   jax.ShapeDtypeStruct((P, 1), f32),
                   jax.ShapeDtypeStruct((P, 1), f32)],
    )

    def prep(coef):
        return pl.pallas_call(
            functools.partial(_prep_body, coef), grid=grid,
            in_specs=[row, row, scat_s, den_s, scal, full_w, vec, vec, vec],
            out_specs=[row, row, row, col, col],
            out_shape=[jax.ShapeDtypeStruct((P, D), f32)] * 3
            + [jax.ShapeDtypeStruct((P, 1), f32)] * 2,
        )

    final = pl.pallas_call(
        _final_body, grid=grid,
        in_specs=[row, row, scat_s, den_s, scal, vec, row, row, row, full_w, vec],
        out_specs=row,
        out_shape=jax.ShapeDtypeStruct((P, D), f32),
    )
    return prep1, prep, final


def _make_sc_stage(P, D, Epad):
    ET = Epad // NW
    NB = ET // K
    assert NB % 6 == 0
    RPT = P // NTILES
    assert RPT % K == 0
    NZ = RPT // K
    mesh = plsc.VectorSubcoreMesh(core_axis_name="c", subcore_axis_name="s")
    f32 = jnp.float32

    @functools.partial(
        pl.kernel, mesh=mesh,
        compiler_params=pltpu.CompilerParams(needs_layout_passes=False),
        out_type=[jax.ShapeDtypeStruct((NSC, P, D), f32),
                  jax.ShapeDtypeStruct((NSC, P), f32)],
        scratch_types=[
            pltpu.VMEM((P,), f32),
            pltpu.VMEM((P,), f32),
            tuple(pltpu.VMEM((K,), jnp.int32) for _ in range(3)),
            tuple(pltpu.VMEM((K,), jnp.int32) for _ in range(3)),
            tuple(pltpu.VMEM((3 * K,), jnp.int32) for _ in range(2)),
            tuple(pltpu.VMEM((3 * K,), jnp.int32) for _ in range(2)),
            tuple(pltpu.VMEM((K, D), f32) for _ in range(3)),
            tuple(pltpu.VMEM((K,), f32) for _ in range(3)),
            pltpu.VMEM_SHARED((P, D), f32),
            pltpu.VMEM_SHARED((P,), f32),
            tuple(pltpu.SemaphoreType.DMA for _ in range(2)),
            tuple(pltpu.SemaphoreType.DMA for _ in range(3)),
            tuple(pltpu.SemaphoreType.DMA for _ in range(3)),
            tuple(pltpu.SemaphoreType.DMA for _ in range(3)),
        ],
    )
    def sc_stage(src_h, dst_h, h_h, asv_h, adv_h, scat_o, den_o,
                 as_v, ad_v, srcb, dstb, csrc, cdst, rows, exb, scat_s, den_s,
                 sem_c, sem_g, sem_s, sem_d):
        core = lax.axis_index("c")
        sub = lax.axis_index("s")
        wid = core * NTILES + sub

        pltpu.sync_copy(asv_h, as_v)
        pltpu.sync_copy(adv_h, ad_v)

        def zrow(i, _):
            for c in range(D // 16):
                rows[0][i, pl.ds(c * 16, 16)] = jnp.zeros((16,), f32)
            return _
        lax.fori_loop(0, K, zrow, None)
        for c in range(K // 16):
            exb[0][pl.ds(c * 16, 16)] = jnp.zeros((16,), f32)

        zbase = sub * RPT

        def zcopy(j, _):
            pltpu.sync_copy(rows[0], scat_s.at[pl.ds(zbase + j * K, K)])
            pltpu.sync_copy(exb[0], den_s.at[pl.ds(zbase + j * K, K)])
            return _
        lax.fori_loop(0, NZ, zcopy, None)

        def amax_body(i, acc):
            return jnp.maximum(acc, as_v[pl.ds(i * 16, 16)])
        acc = lax.fori_loop(0, P // 16, amax_body, jnp.zeros((16,), f32))
        big_a = plsc.cummax(acc)[15]

        plsc.subcore_barrier()

        ebase = wid * ET
        CH = 3 * K

        def issue_chunk(p, t):
            off = ebase + t * CH
            pltpu.async_copy(src_h.at[pl.ds(off, CH)], csrc[p], sem_c[p])
            pltpu.async_copy(dst_h.at[pl.ds(off, CH)], cdst[p], sem_c[p])

        def wait_chunk(p, t):
            off = ebase + t * CH
            pltpu.make_async_copy(src_h.at[pl.ds(off, CH)], csrc[p],
                                  sem_c[p]).wait()
            pltpu.make_async_copy(dst_h.at[pl.ds(off, CH)], cdst[p],
                                  sem_c[p]).wait()

        def copy_idx(slot, p, loff):
            for g in range(K // 16):
                sl = pl.ds(g * 16, 16)
                cl = pl.ds(loff + g * 16, 16)
                srcb[slot][sl] = csrc[p][cl]
                dstb[slot][sl] = cdst[p][cl]

        def issue_gather(slot):
            pltpu.async_copy(h_h.at[srcb[slot]], rows[slot], sem_g[slot])

        def wait_gather(slot):
            pltpu.make_async_copy(h_h.at[srcb[slot]], rows[slot],
                                  sem_g[slot]).wait()

        pltpu.sync_copy(src_h.at[pl.ds(ebase, CH)], csrc[0])
        pltpu.sync_copy(dst_h.at[pl.ds(ebase, CH)], cdst[0])
        for b in range(2):
            copy_idx(b, 0, b * K)
            issue_gather(b)

        def six(i6, _):
            sdescs = []
            for tu in range(2):
                t = 2 * i6 + tu
                issue_chunk(1 - tu, t + 1)
                for u in range(3):
                    b = 3 * t + u
                    slot = u
                    j = 3 * tu + u
                    wait_gather(slot)

                    def grp(g, _, slot=slot):
                        sl = pl.ds(g * 16, 16)
                        srcg = srcb[slot][sl]
                        dstg = dstb[slot][sl]
                        s16 = plsc.load_gather(as_v, [srcg])
                        d16 = plsc.load_gather(ad_v, [dstg])
                        e = s16 + d16
                        e = jnp.where(e > 0.0, e, 0.2 * e)
                        mt = jnp.maximum(big_a + d16, 0.0)
                        ex = jnp.exp(e - mt)
                        exb[slot][sl] = ex
                        for i in range(16):
                            w = ex[i]
                            jj = g * 16 + i
                            for c in range(D // 16):
                                cs = pl.ds(c * 16, 16)
                                rows[slot][jj, cs] = rows[slot][jj, cs] * w
                        return _
                    lax.fori_loop(0, K // 16, grp, None)
                    s = pltpu.async_copy(rows[slot], scat_s.at[dstb[slot]],
                                         sem_s[slot], add=True)
                    d = pltpu.async_copy(exb[slot], den_s.at[dstb[slot]],
                                         sem_d[slot], add=True)
                    sdescs.append((s, d))

                    if j >= 1:
                        ps, pd = sdescs[j - 1]
                        ps.wait()
                        pd.wait()
                    nslot = (u + 2) % 3
                    if u == 0:
                        copy_idx(nslot, tu, 2 * K)
                    elif u == 1:
                        wait_chunk(1 - tu, t + 1)
                        copy_idx(nslot, 1 - tu, 0)
                    else:
                        copy_idx(nslot, 1 - tu, K)
                    issue_gather(nslot)
            ls, ld = sdescs[5]
            ls.wait()
            ld.wait()
            return _
        lax.fori_loop(0, NB // 6, six, None)

        for b in (NB, NB + 1):
            wait_gather(b % 3)

        plsc.subcore_barrier()

        pltpu.sync_copy(scat_s.at[pl.ds(zbase, RPT)],
                        scat_o.at[core, pl.ds(zbase, RPT)])
        pltpu.sync_copy(den_s.at[pl.ds(zbase, RPT)],
                        den_o.at[core, pl.ds(zbase, RPT)])

    return sc_stage


def kernel(x, edge_index, W, att_src, att_dst, bias, alpha_train, W_out, b_out):
    N, D = x.shape
    E = edge_index.shape[1]
    P = _round_up(N, 5120)

    ei = edge_index.astype(jnp.int32)
    loops = jnp.arange(N, dtype=jnp.int32)
    ETOT = E + N
    Epad = _round_up(ETOT, NW * 6 * K)
    fill = jnp.full((Epad + 3 * K - ETOT,), N, dtype=jnp.int32)
    src = jnp.concatenate([ei[0], loops, fill])
    dst = jnp.concatenate([ei[1], loops, fill])

    xp = jnp.pad(x, ((0, P - N), (0, 0)))
    alpha2d = alpha_train.reshape(1, 1)

    prep1, prep, final = _make_tc_calls(P, D)
    sc_stage = _make_sc_stage(P, D, Epad)

    h1, as1, ad1 = prep1(xp, W, att_src, att_dst)
    sc1, dn1 = sc_stage(src, dst, h1, as1.reshape(P), ad1.reshape(P))

    k1, z2, h2, as2, ad2 = prep(0.5 * EPS)(xp, xp, sc1, dn1, alpha2d, W,
                                           att_src, att_dst, bias)
    sc2, dn2 = sc_stage(src, dst, h2, as2.reshape(P), ad2.reshape(P))

    k2, z3, h3, as3, ad3 = prep(0.5 * EPS)(xp, z2, sc2, dn2, alpha2d, W,
                                           att_src, att_dst, bias)
    sc3, dn3 = sc_stage(src, dst, h3, as3.reshape(P), ad3.reshape(P))

    k3, z4, h4, as4, ad4 = prep(EPS)(xp, z3, sc3, dn3, alpha2d, W,
                                     att_src, att_dst, bias)
    sc4, dn4 = sc_stage(src, dst, h4, as4.reshape(P), ad4.reshape(P))

    out = final(xp, z4, sc4, dn4, alpha2d, bias, k1, k2, k3, W_out, b_out)
    return out[:N]


# speedup vs baseline: 37.1772x; 1.0139x over previous
import functools

import jax
import jax.numpy as jnp
from jax import lax
from jax.experimental import pallas as pl
from jax.experimental.pallas import tpu as pltpu
from jax.experimental.pallas import tpu_sc as plsc

EPS = 0.1
RB = 512
K = 64
NSC = 2
NTILES = 16
NW = NSC * NTILES


def _round_up(v, m):
    return (v + m - 1) // m * m


def _dense_tail(h_blk, att_s, att_d):
    a_s = jnp.sum(h_blk * att_s[None, :], axis=1, keepdims=True)
    a_d = jnp.sum(h_blk * att_d[None, :], axis=1, keepdims=True)
    return a_s, a_d


def _prep1_body(x_ref, w_ref, as_ref, ad_ref, h_ref, asrc_ref, adst_ref):
    h = jnp.dot(x_ref[...], w_ref[...], preferred_element_type=jnp.float32)
    h_ref[...] = h
    a_s, a_d = _dense_tail(h, as_ref[...], ad_ref[...])
    asrc_ref[...] = a_s
    adst_ref[...] = a_d


def _gat_from_parts(scat, den, bias):
    num = scat[0] + scat[1]
    dn = den[0] + den[1] + 1e-16
    return num / dn[:, None] + bias[None, :]


def _prep_body(coef, x_ref, zp_ref, scat_ref, den_ref, alpha_ref, w_ref,
               as_ref, ad_ref, bias_ref,
               k_ref, z_ref, h_ref, asrc_ref, adst_ref):
    a = jax.nn.sigmoid(alpha_ref[0, 0])
    gat = _gat_from_parts(scat_ref[...], den_ref[...], bias_ref[...])
    k = a * (gat - zp_ref[...])
    k_ref[...] = k
    z = x_ref[...] + coef * k
    z_ref[...] = z
    h = jnp.dot(z, w_ref[...], preferred_element_type=jnp.float32)
    h_ref[...] = h
    a_s, a_d = _dense_tail(h, as_ref[...], ad_ref[...])
    asrc_ref[...] = a_s
    adst_ref[...] = a_d


def _final_body(x_ref, zp_ref, scat_ref, den_ref, alpha_ref, bias_ref,
                k1_ref, k2_ref, k3_ref, wout_ref, bout_ref, out_ref):
    a = jax.nn.sigmoid(alpha_ref[0, 0])
    gat = _gat_from_parts(scat_ref[...], den_ref[...], bias_ref[...])
    k4 = a * (gat - zp_ref[...])
    z = x_ref[...] + (EPS / 6.0) * (k1_ref[...] + 2.0 * k2_ref[...]
                                    + 2.0 * k3_ref[...] + k4)
    y = jnp.tanh(z)
    out_ref[...] = jnp.dot(y, wout_ref[...], preferred_element_type=jnp.float32) \
        + bout_ref[...][None, :]


def _mat_spec(i_map=lambda i: (i, 0), blk=None):
    return pl.BlockSpec(blk, i_map)


def _make_tc_calls(P, D):
    grid = (P // RB,)
    row = pl.BlockSpec((RB, D), lambda i: (i, 0))
    full_w = pl.BlockSpec((D, D), lambda i: (0, 0))
    vec = pl.BlockSpec((D,), lambda i: (0,))
    col = pl.BlockSpec((RB, 1), lambda i: (i, 0))
    scat_s = pl.BlockSpec((NSC, RB, D), lambda i: (0, i, 0))
    den_s = pl.BlockSpec((NSC, RB), lambda i: (0, i))
    scal = pl.BlockSpec((1, 1), lambda i: (0, 0))

    f32 = jnp.float32
    prep1 = pl.pallas_call(
        _prep1_body, grid=grid,
        in_specs=[row, full_w, vec, vec],
        out_specs=[row, col, col],
        out_shape=[jax.ShapeDtypeStruct((P, D), f32),
                   jax.ShapeDtypeStruct((P, 1), f32),
                   jax.ShapeDtypeStruct((P, 1), f32)],
    )

    def prep(coef):
        return pl.pallas_call(
            functools.partial(_prep_body, coef), grid=grid,
            in_specs=[row, row, scat_s, den_s, scal, full_w, vec, vec, vec],
            out_specs=[row, row, row, col, col],
            out_shape=[jax.ShapeDtypeStruct((P, D), f32)] * 3
            + [jax.ShapeDtypeStruct((P, 1), f32)] * 2,
        )

    final = pl.pallas_call(
        _final_body, grid=grid,
        in_specs=[row, row, scat_s, den_s, scal, vec, row, row, row, full_w, vec],
        out_specs=row,
        out_shape=jax.ShapeDtypeStruct((P, D), f32),
    )
    return prep1, prep, final


def _make_sc_stage(P, D, Epad):
    ET = Epad // NW
    NB = ET // K
    assert NB % 6 == 0
    RPT = P // NTILES
    assert RPT % K == 0
    NZ = RPT // K
    mesh = plsc.VectorSubcoreMesh(core_axis_name="c", subcore_axis_name="s")
    f32 = jnp.float32

    @functools.partial(
        pl.kernel, mesh=mesh,
        compiler_params=pltpu.CompilerParams(needs_layout_passes=False),
        out_type=[jax.ShapeDtypeStruct((NSC, P, D), f32),
                  jax.ShapeDtypeStruct((NSC, P), f32)],
        scratch_types=[
            pltpu.VMEM((P,), f32),
            pltpu.VMEM((P,), f32),
            tuple(pltpu.VMEM((K,), jnp.int32) for _ in range(3)),
            tuple(pltpu.VMEM((K,), jnp.int32) for _ in range(3)),
            tuple(pltpu.VMEM((3 * K,), jnp.int32) for _ in range(2)),
            tuple(pltpu.VMEM((3 * K,), jnp.int32) for _ in range(2)),
            tuple(pltpu.VMEM((K, D), f32) for _ in range(3)),
            tuple(pltpu.VMEM((K,), f32) for _ in range(3)),
            pltpu.VMEM_SHARED((P, D), f32),
            pltpu.VMEM_SHARED((P,), f32),
            tuple(pltpu.SemaphoreType.DMA for _ in range(2)),
            tuple(pltpu.SemaphoreType.DMA for _ in range(3)),
            tuple(pltpu.SemaphoreType.DMA for _ in range(3)),
            tuple(pltpu.SemaphoreType.DMA for _ in range(3)),
        ],
    )
    def sc_stage(src_h, dst_h, h_h, asv_h, adv_h, scat_o, den_o,
                 as_v, ad_v, srcb, dstb, csrc, cdst, rows, exb, scat_s, den_s,
                 sem_c, sem_g, sem_s, sem_d):
        core = lax.axis_index("c")
        sub = lax.axis_index("s")
        wid = core * NTILES + sub

        pltpu.sync_copy(asv_h, as_v)
        pltpu.sync_copy(adv_h, ad_v)

        def zrow(i, _):
            for c in range(D // 16):
                rows[0][i, pl.ds(c * 16, 16)] = jnp.zeros((16,), f32)
            return _
        lax.fori_loop(0, K, zrow, None)
        for c in range(K // 16):
            exb[0][pl.ds(c * 16, 16)] = jnp.zeros((16,), f32)

        zbase = sub * RPT

        for j in range(NZ):
            pltpu.async_copy(rows[0], scat_s.at[pl.ds(zbase + j * K, K)],
                             sem_s[j % 3])
            pltpu.async_copy(exb[0], den_s.at[pl.ds(zbase + j * K, K)],
                             sem_d[j % 3])

        def amax_body(i, acc):
            return jnp.maximum(acc, as_v[pl.ds(i * 16, 16)])
        acc = lax.fori_loop(0, P // 16, amax_body, jnp.zeros((16,), f32))
        big_a = plsc.cummax(acc)[15]

        for j in range(NZ):
            pltpu.make_async_copy(rows[0], scat_s.at[pl.ds(zbase + j * K, K)],
                                  sem_s[j % 3]).wait()
            pltpu.make_async_copy(exb[0], den_s.at[pl.ds(zbase + j * K, K)],
                                  sem_d[j % 3]).wait()

        plsc.subcore_barrier()

        ebase = wid * ET
        CH = 3 * K

        def issue_chunk(p, t):
            off = ebase + t * CH
            pltpu.async_copy(src_h.at[pl.ds(off, CH)], csrc[p], sem_c[p])
            pltpu.async_copy(dst_h.at[pl.ds(off, CH)], cdst[p], sem_c[p])

        def wait_chunk(p, t):
            off = ebase + t * CH
            pltpu.make_async_copy(src_h.at[pl.ds(off, CH)], csrc[p],
                                  sem_c[p]).wait()
            pltpu.make_async_copy(dst_h.at[pl.ds(off, CH)], cdst[p],
                                  sem_c[p]).wait()

        def copy_idx(slot, p, loff):
            for g in range(K // 16):
                sl = pl.ds(g * 16, 16)
                cl = pl.ds(loff + g * 16, 16)
                srcb[slot][sl] = csrc[p][cl]
                dstb[slot][sl] = cdst[p][cl]

        def issue_gather(slot):
            pltpu.async_copy(h_h.at[srcb[slot]], rows[slot], sem_g[slot])

        def wait_gather(slot):
            pltpu.make_async_copy(h_h.at[srcb[slot]], rows[slot],
                                  sem_g[slot]).wait()

        pltpu.sync_copy(src_h.at[pl.ds(ebase, CH)], csrc[0])
        pltpu.sync_copy(dst_h.at[pl.ds(ebase, CH)], cdst[0])
        for b in range(2):
            copy_idx(b, 0, b * K)
            issue_gather(b)

        def six(i6, _):
            sdescs = []
            for tu in range(2):
                t = 2 * i6 + tu
                issue_chunk(1 - tu, t + 1)
                for u in range(3):
                    b = 3 * t + u
                    slot = u
                    j = 3 * tu + u
                    wait_gather(slot)

                    def grp(g, _, slot=slot):
                        sl = pl.ds(g * 16, 16)
                        srcg = srcb[slot][sl]
                        dstg = dstb[slot][sl]
                        s16 = plsc.load_gather(as_v, [srcg])
                        d16 = plsc.load_gather(ad_v, [dstg])
                        e = s16 + d16
                        e = jnp.where(e > 0.0, e, 0.2 * e)
                        mt = jnp.maximum(big_a + d16, 0.0)
                        ex = jnp.exp(e - mt)
                        exb[slot][sl] = ex
                        for i in range(16):
                            w = ex[i]
                            jj = g * 16 + i
                            for c in range(D // 16):
                                cs = pl.ds(c * 16, 16)
                                rows[slot][jj, cs] = rows[slot][jj, cs] * w
                        return _
                    lax.fori_loop(0, K // 16, grp, None)
                    s = pltpu.async_copy(rows[slot], scat_s.at[dstb[slot]],
                                         sem_s[slot], add=True)
                    d = pltpu.async_copy(exb[slot], den_s.at[dstb[slot]],
                                         sem_d[slot], add=True)
                    sdescs.append((s, d))

                    if j >= 1:
                        ps, pd = sdescs[j - 1]
                        ps.wait()
                        pd.wait()
                    nslot = (u + 2) % 3
                    if u == 0:
                        copy_idx(nslot, tu, 2 * K)
                    elif u == 1:
                        wait_chunk(1 - tu, t + 1)
                        copy_idx(nslot, 1 - tu, 0)
                    else:
                        copy_idx(nslot, 1 - tu, K)
                    issue_gather(nslot)
            ls, ld = sdescs[5]
            ls.wait()
            ld.wait()
            return _
        lax.fori_loop(0, NB // 6, six, None)

        for b in (NB, NB + 1):
            wait_gather(b % 3)

        plsc.subcore_barrier()

        pltpu.sync_copy(scat_s.at[pl.ds(zbase, RPT)],
                        scat_o.at[core, pl.ds(zbase, RPT)])
        pltpu.sync_copy(den_s.at[pl.ds(zbase, RPT)],
                        den_o.at[core, pl.ds(zbase, RPT)])

    return sc_stage


def kernel(x, edge_index, W, att_src, att_dst, bias, alpha_train, W_out, b_out):
    N, D = x.shape
    E = edge_index.shape[1]
    P = _round_up(N, 5120)

    ei = edge_index.astype(jnp.int32)
    loops = jnp.arange(N, dtype=jnp.int32)
    ETOT = E + N
    Epad = _round_up(ETOT, NW * 6 * K)
    fill = jnp.full((Epad + 3 * K - ETOT,), N, dtype=jnp.int32)
    src = jnp.concatenate([ei[0], loops, fill])
    dst = jnp.concatenate([ei[1], loops, fill])

    xp = jnp.pad(x, ((0, P - N), (0, 0)))
    alpha2d = alpha_train.reshape(1, 1)

    prep1, prep, final = _make_tc_calls(P, D)
    sc_stage = _make_sc_stage(P, D, Epad)

    h1, as1, ad1 = prep1(xp, W, att_src, att_dst)
    sc1, dn1 = sc_stage(src, dst, h1, as1.reshape(P), ad1.reshape(P))

    k1, z2, h2, as2, ad2 = prep(0.5 * EPS)(xp, xp, sc1, dn1, alpha2d, W,
                                           att_src, att_dst, bias)
    sc2, dn2 = sc_stage(src, dst, h2, as2.reshape(P), ad2.reshape(P))

    k2, z3, h3, as3, ad3 = prep(0.5 * EPS)(xp, z2, sc2, dn2, alpha2d, W,
                                           att_src, att_dst, bias)
    sc3, dn3 = sc_stage(src, dst, h3, as3.reshape(P), ad3.reshape(P))

    k3, z4, h4, as4, ad4 = prep(EPS)(xp, z3, sc3, dn3, alpha2d, W,
                                     att_src, att_dst, bias)
    sc4, dn4 = sc_stage(src, dst, h4, as4.reshape(P), ad4.reshape(P))

    out = final(xp, z4, sc4, dn4, alpha2d, bias, k1, k2, k3, W_out, b_out)
    return out[:N]
